# Initial kernel scaffold; baseline (speedup 1.0000x reference)
import jax
import jax.numpy as jnp
from jax.experimental import pallas as pl


def kernel(x, nbr, W_ih, W_hh, b_ih, b_hh, W_self1, W_neigh1, b1, W_pool, b_pool, W_self2, W_neigh2, b2):
    raise NotImplementedError("write your pallas kernel here")


# speedup vs baseline: 1.3969x; 1.3969x over previous
import functools

import jax
import jax.numpy as jnp
from jax import lax
from jax.experimental import pallas as pl
from jax.experimental.pallas import tpu as pltpu
from jax.experimental.pallas import tpu_sc as plsc

N = 10000
D = 32
F_IN = 128
HID = 256
NCLS = 16

NP = 10240
BLK = 512
NB = NP // BLK
CHUNK = 128
NSC = 2
NTILE = 16
NW = NSC * NTILE


def _sc_gather(table, idx3, feat):
    _, n_chunks, _ = idx3.shape
    per_w = n_chunks * CHUNK
    rows_total = NW * per_w
    mesh = plsc.VectorSubcoreMesh(core_axis_name="c", subcore_axis_name="s")

    @functools.partial(
        pl.kernel,
        mesh=mesh,
        out_type=jax.ShapeDtypeStruct((rows_total, feat), jnp.float32),
        scratch_types=[
            pltpu.VMEM((n_chunks, CHUNK), jnp.int32),
            pltpu.VMEM((CHUNK, feat), jnp.float32),
            pltpu.SemaphoreType.DMA,
        ],
    )
    def gk(table_hbm, idx_hbm, out_hbm, idx_v, rows_v, sem):
        wid = lax.axis_index("s") * NSC + lax.axis_index("c")
        base = wid * per_w
        pltpu.sync_copy(idx_hbm.at[wid], idx_v)

        def body(j, carry):
            pltpu.async_copy(table_hbm.at[idx_v.at[j]], rows_v, sem).wait()
            pltpu.sync_copy(rows_v, out_hbm.at[pl.ds(base + j * CHUNK, CHUNK)])
            return carry

        lax.fori_loop(0, n_chunks, body, 0)

    return gk(table, idx3)


def _conv1(m, xp, W_ihT, W_hhT, bias, W_self1, W_neigh1, b1):
    def body(m_ref, x_ref, wih_ref, whh_ref, b_ref, ws_ref, wn_ref, b1_ref,
             out_ref, h_s, c_s):
        d = pl.program_id(1)

        @pl.when(d == 0)
        def _():
            h_s[...] = jnp.zeros_like(h_s)
            c_s[...] = jnp.zeros_like(c_s)

        xt = m_ref[0]
        gates = (jnp.dot(xt, wih_ref[...], preferred_element_type=jnp.float32)
                 + jnp.dot(h_s[...], whh_ref[...], preferred_element_type=jnp.float32)
                 + b_ref[...])
        gi = jax.nn.sigmoid(gates[:, 0:F_IN])
        gf = jax.nn.sigmoid(gates[:, F_IN:2 * F_IN])
        gg = jnp.tanh(gates[:, 2 * F_IN:3 * F_IN])
        go = jax.nn.sigmoid(gates[:, 3 * F_IN:4 * F_IN])
        c = gf * c_s[...] + gi * gg
        h = go * jnp.tanh(c)
        c_s[...] = c
        h_s[...] = h

        @pl.when(d == D - 1)
        def _():
            out_ref[...] = jax.nn.relu(
                jnp.dot(x_ref[...], ws_ref[...], preferred_element_type=jnp.float32)
                + jnp.dot(h, wn_ref[...], preferred_element_type=jnp.float32)
                + b1_ref[...])

    return pl.pallas_call(
        body,
        grid=(NB, D),
        in_specs=[
            pl.BlockSpec((1, BLK, F_IN), lambda i, d: (d, i, 0)),
            pl.BlockSpec((BLK, F_IN), lambda i, d: (i, 0)),
            pl.BlockSpec((F_IN, 4 * F_IN), lambda i, d: (0, 0)),
            pl.BlockSpec((F_IN, 4 * F_IN), lambda i, d: (0, 0)),
            pl.BlockSpec((1, 4 * F_IN), lambda i, d: (0, 0)),
            pl.BlockSpec((F_IN, HID), lambda i, d: (0, 0)),
            pl.BlockSpec((F_IN, HID), lambda i, d: (0, 0)),
            pl.BlockSpec((1, HID), lambda i, d: (0, 0)),
        ],
        out_specs=pl.BlockSpec((BLK, HID), lambda i, d: (i, 0)),
        out_shape=jax.ShapeDtypeStruct((NP, HID), jnp.float32),
        scratch_shapes=[
            pltpu.VMEM((BLK, F_IN), jnp.float32),
            pltpu.VMEM((BLK, F_IN), jnp.float32),
        ],
    )(m, xp, W_ihT, W_hhT, bias, W_self1, W_neigh1, b1)


def _conv2(m2, h, W_pool, b_pool, W_self2, W_neigh2, b2):
    def body(m_ref, h_ref, wp_ref, bp_ref, ws_ref, wn_ref, b2_ref,
             out_ref, mx_s):
        d = pl.program_id(1)
        t = jax.nn.relu(
            jnp.dot(m_ref[0], wp_ref[...], preferred_element_type=jnp.float32)
            + bp_ref[...])
        prev = jnp.where(d == 0, jnp.zeros_like(t), mx_s[...])
        mx = jnp.maximum(t, prev)
        mx_s[...] = mx

        @pl.when(d == D - 1)
        def _():
            out_ref[...] = (
                jnp.dot(h_ref[...], ws_ref[...], preferred_element_type=jnp.float32)
                + jnp.dot(mx, wn_ref[...], preferred_element_type=jnp.float32)
                + b2_ref[...])

    return pl.pallas_call(
        body,
        grid=(NB, D),
        in_specs=[
            pl.BlockSpec((1, BLK, HID), lambda i, d: (d, i, 0)),
            pl.BlockSpec((BLK, HID), lambda i, d: (i, 0)),
            pl.BlockSpec((HID, HID), lambda i, d: (0, 0)),
            pl.BlockSpec((1, HID), lambda i, d: (0, 0)),
            pl.BlockSpec((HID, NCLS), lambda i, d: (0, 0)),
            pl.BlockSpec((HID, NCLS), lambda i, d: (0, 0)),
            pl.BlockSpec((1, NCLS), lambda i, d: (0, 0)),
        ],
        out_specs=pl.BlockSpec((BLK, NCLS), lambda i, d: (i, 0)),
        out_shape=jax.ShapeDtypeStruct((NP, NCLS), jnp.float32),
        scratch_shapes=[pltpu.VMEM((BLK, HID), jnp.float32)],
    )(m2, h, W_pool, b_pool, W_self2, W_neigh2, b2)


def kernel(x, nbr, W_ih, W_hh, b_ih, b_hh, W_self1, W_neigh1, b1,
           W_pool, b_pool, W_self2, W_neigh2, b2):
    nbr = nbr.astype(jnp.int32)
    xp = jnp.pad(x, ((0, NP - N), (0, 0)))
    idx = jnp.pad(nbr.T, ((0, 0), (0, NP - N))).reshape(-1)
    idx3 = idx.reshape(NW, (D * NP) // (NW * CHUNK), CHUNK)

    m = _sc_gather(x, idx3, F_IN).reshape(D, NP, F_IN)
    bias = (b_ih + b_hh).reshape(1, 4 * F_IN)
    h = _conv1(m, xp, W_ih.T, W_hh.T, bias, W_self1, W_neigh1,
               b1.reshape(1, HID))
    m2 = _sc_gather(h, idx3, HID).reshape(D, NP, HID)
    out = _conv2(m2, h, W_pool, b_pool.reshape(1, HID), W_self2, W_neigh2,
                 b2.reshape(1, NCLS))
    return out[:N]


# speedup vs baseline: 1.4285x; 1.0226x over previous
import functools

import jax
import jax.numpy as jnp
from jax import lax
from jax.experimental import pallas as pl
from jax.experimental.pallas import tpu as pltpu
from jax.experimental.pallas import tpu_sc as plsc

N = 10000
D = 32
F_IN = 128
HID = 256
NCLS = 16

NP = 10240
BLK = 512
NB = NP // BLK
CHUNK = 128
NSC = 2
NTILE = 16
NW = NSC * NTILE


def _sc_gather(table, idx3, feat):
    _, n_chunks, _ = idx3.shape
    per_w = n_chunks * CHUNK
    rows_total = NW * per_w
    mesh = plsc.VectorSubcoreMesh(core_axis_name="c", subcore_axis_name="s")

    @functools.partial(
        pl.kernel,
        mesh=mesh,
        out_type=jax.ShapeDtypeStruct((rows_total, feat), jnp.float32),
        scratch_types=[
            pltpu.VMEM((n_chunks, CHUNK), jnp.int32),
            pltpu.VMEM((CHUNK, feat), jnp.float32),
            pltpu.SemaphoreType.DMA,
        ],
    )
    def gk(table_hbm, idx_hbm, out_hbm, idx_v, rows_v, sem):
        wid = lax.axis_index("s") * NSC + lax.axis_index("c")
        base = wid * per_w
        pltpu.sync_copy(idx_hbm.at[wid], idx_v)

        def body(j, carry):
            pltpu.async_copy(table_hbm.at[idx_v.at[j]], rows_v, sem).wait()
            pltpu.sync_copy(rows_v, out_hbm.at[pl.ds(base + j * CHUNK, CHUNK)])
            return carry

        lax.fori_loop(0, n_chunks, body, 0)

    return gk(table, idx3)


def _conv1(m, xp, W_ihT, W_hhT, bias, W_self1, W_neigh1, b1, W_pool, b_pool):
    def body(m_ref, x_ref, wih_ref, whh_ref, b_ref, ws_ref, wn_ref, b1_ref,
             wp_ref, bp_ref, out_ref, q_ref, h_s, c_s):
        d = pl.program_id(1)

        @pl.when(d == 0)
        def _():
            h_s[...] = jnp.zeros_like(h_s)
            c_s[...] = jnp.zeros_like(c_s)

        xt = m_ref[0]
        gates = (jnp.dot(xt, wih_ref[...], preferred_element_type=jnp.float32)
                 + jnp.dot(h_s[...], whh_ref[...], preferred_element_type=jnp.float32)
                 + b_ref[...])
        gi = jax.nn.sigmoid(gates[:, 0:F_IN])
        gf = jax.nn.sigmoid(gates[:, F_IN:2 * F_IN])
        gg = jnp.tanh(gates[:, 2 * F_IN:3 * F_IN])
        go = jax.nn.sigmoid(gates[:, 3 * F_IN:4 * F_IN])
        c = gf * c_s[...] + gi * gg
        h = go * jnp.tanh(c)
        c_s[...] = c
        h_s[...] = h

        @pl.when(d == D - 1)
        def _():
            hh = jax.nn.relu(
                jnp.dot(x_ref[...], ws_ref[...], preferred_element_type=jnp.float32)
                + jnp.dot(h, wn_ref[...], preferred_element_type=jnp.float32)
                + b1_ref[...])
            out_ref[...] = hh
            q_ref[...] = jax.nn.relu(
                jnp.dot(hh, wp_ref[...], preferred_element_type=jnp.float32)
                + bp_ref[...])

    return pl.pallas_call(
        body,
        grid=(NB, D),
        in_specs=[
            pl.BlockSpec((1, BLK, F_IN), lambda i, d: (d, i, 0)),
            pl.BlockSpec((BLK, F_IN), lambda i, d: (i, 0)),
            pl.BlockSpec((F_IN, 4 * F_IN), lambda i, d: (0, 0)),
            pl.BlockSpec((F_IN, 4 * F_IN), lambda i, d: (0, 0)),
            pl.BlockSpec((1, 4 * F_IN), lambda i, d: (0, 0)),
            pl.BlockSpec((F_IN, HID), lambda i, d: (0, 0)),
            pl.BlockSpec((F_IN, HID), lambda i, d: (0, 0)),
            pl.BlockSpec((1, HID), lambda i, d: (0, 0)),
            pl.BlockSpec((HID, HID), lambda i, d: (0, 0)),
            pl.BlockSpec((1, HID), lambda i, d: (0, 0)),
        ],
        out_specs=[
            pl.BlockSpec((BLK, HID), lambda i, d: (i, 0)),
            pl.BlockSpec((BLK, HID), lambda i, d: (i, 0)),
        ],
        out_shape=[
            jax.ShapeDtypeStruct((NP, HID), jnp.float32),
            jax.ShapeDtypeStruct((NP, HID), jnp.float32),
        ],
        scratch_shapes=[
            pltpu.VMEM((BLK, F_IN), jnp.float32),
            pltpu.VMEM((BLK, F_IN), jnp.float32),
        ],
    )(m, xp, W_ihT, W_hhT, bias, W_self1, W_neigh1, b1, W_pool, b_pool)


def _conv2(m2, h, W_self2, W_neigh2, b2):
    def body(m_ref, h_ref, ws_ref, wn_ref, b2_ref, out_ref, mx_s):
        d = pl.program_id(1)
        t = m_ref[0]
        prev = jnp.where(d == 0, jnp.zeros_like(t), mx_s[...])
        mx = jnp.maximum(t, prev)
        mx_s[...] = mx

        @pl.when(d == D - 1)
        def _():
            out_ref[...] = (
                jnp.dot(h_ref[...], ws_ref[...], preferred_element_type=jnp.float32)
                + jnp.dot(mx, wn_ref[...], preferred_element_type=jnp.float32)
                + b2_ref[...])

    return pl.pallas_call(
        body,
        grid=(NB, D),
        in_specs=[
            pl.BlockSpec((1, BLK, HID), lambda i, d: (d, i, 0)),
            pl.BlockSpec((BLK, HID), lambda i, d: (i, 0)),
            pl.BlockSpec((HID, NCLS), lambda i, d: (0, 0)),
            pl.BlockSpec((HID, NCLS), lambda i, d: (0, 0)),
            pl.BlockSpec((1, NCLS), lambda i, d: (0, 0)),
        ],
        out_specs=pl.BlockSpec((BLK, NCLS), lambda i, d: (i, 0)),
        out_shape=jax.ShapeDtypeStruct((NP, NCLS), jnp.float32),
        scratch_shapes=[pltpu.VMEM((BLK, HID), jnp.float32)],
    )(m2, h, W_self2, W_neigh2, b2)


def kernel(x, nbr, W_ih, W_hh, b_ih, b_hh, W_self1, W_neigh1, b1,
           W_pool, b_pool, W_self2, W_neigh2, b2):
    nbr = nbr.astype(jnp.int32)
    xp = jnp.pad(x, ((0, NP - N), (0, 0)))
    idx = jnp.pad(nbr.T, ((0, 0), (0, NP - N))).reshape(-1)
    idx3 = idx.reshape(NW, (D * NP) // (NW * CHUNK), CHUNK)

    m = _sc_gather(x, idx3, F_IN).reshape(D, NP, F_IN)
    bias = (b_ih + b_hh).reshape(1, 4 * F_IN)
    h, q = _conv1(m, xp, W_ih.T, W_hh.T, bias, W_self1, W_neigh1,
                  b1.reshape(1, HID), W_pool, b_pool.reshape(1, HID))
    m2 = _sc_gather(q, idx3, HID).reshape(D, NP, HID)
    out = _conv2(m2, h, W_self2, W_neigh2, b2.reshape(1, NCLS))
    return out[:N]


# speedup vs baseline: 1.4934x; 1.0454x over previous
import functools

import jax
import jax.numpy as jnp
from jax import lax
from jax.experimental import pallas as pl
from jax.experimental.pallas import tpu as pltpu
from jax.experimental.pallas import tpu_sc as plsc

N = 10000
D = 32
F_IN = 128
HID = 256
NCLS = 16

NP = 10240
BLK = 512
NB = NP // BLK
CHUNK = 128
NSC = 2
NTILE = 16
NW = NSC * NTILE


NBUF = 4


def _sc_gather(table, idx3, feat):
    _, n_chunks, chunk = idx3.shape
    per_w = n_chunks * chunk
    rows_total = NW * per_w
    n_iters = n_chunks // NBUF
    mesh = plsc.VectorSubcoreMesh(core_axis_name="c", subcore_axis_name="s")

    @functools.partial(
        pl.kernel,
        mesh=mesh,
        out_type=jax.ShapeDtypeStruct((rows_total, feat), jnp.float32),
        scratch_types=[
            pltpu.VMEM((n_chunks, chunk), jnp.int32),
        ]
        + [pltpu.VMEM((chunk, feat), jnp.float32) for _ in range(NBUF)]
        + [pltpu.SemaphoreType.DMA for _ in range(2 * NBUF)],
    )
    def gk(table_hbm, idx_hbm, out_hbm, idx_v, *rest):
        bufs = rest[:NBUF]
        gsems = rest[NBUF:2 * NBUF]
        osems = rest[2 * NBUF:]
        wid = lax.axis_index("s") * NSC + lax.axis_index("c")
        base = wid * per_w
        pltpu.sync_copy(idx_hbm.at[wid], idx_v)
        for b in range(NBUF):
            pltpu.async_copy(table_hbm.at[idx_v.at[b]], bufs[b], gsems[b])

        def body(k, carry):
            for b in range(NBUF):
                j = k * NBUF + b
                pltpu.make_async_copy(
                    table_hbm.at[idx_v.at[j]], bufs[b], gsems[b]).wait()
                out_slice = out_hbm.at[pl.ds(base + j * chunk, chunk)]
                pltpu.async_copy(bufs[b], out_slice, osems[b])

                @pl.when(k < n_iters - 1)
                def _():
                    pltpu.make_async_copy(bufs[b], out_slice, osems[b]).wait()
                    pltpu.async_copy(
                        table_hbm.at[idx_v.at[j + NBUF]], bufs[b], gsems[b])
            return carry

        lax.fori_loop(0, n_iters, body, 0)
        for b in range(NBUF):
            j = (n_iters - 1) * NBUF + b
            out_slice = out_hbm.at[pl.ds(base + j * chunk, chunk)]
            pltpu.make_async_copy(bufs[b], out_slice, osems[b]).wait()

    return gk(table, idx3)


def _conv1(m, xp, W_ihT, W_hhT, bias, W_self1, W_neigh1, b1, W_pool, b_pool):
    def body(m_ref, x_ref, wih_ref, whh_ref, b_ref, ws_ref, wn_ref, b1_ref,
             wp_ref, bp_ref, out_ref, q_ref, h_s, c_s):
        d = pl.program_id(1)

        @pl.when(d == 0)
        def _():
            h_s[...] = jnp.zeros_like(h_s)
            c_s[...] = jnp.zeros_like(c_s)

        xt = m_ref[0]
        gates = (jnp.dot(xt, wih_ref[...], preferred_element_type=jnp.float32)
                 + jnp.dot(h_s[...], whh_ref[...], preferred_element_type=jnp.float32)
                 + b_ref[...])
        gi = jax.nn.sigmoid(gates[:, 0:F_IN])
        gf = jax.nn.sigmoid(gates[:, F_IN:2 * F_IN])
        gg = jnp.tanh(gates[:, 2 * F_IN:3 * F_IN])
        go = jax.nn.sigmoid(gates[:, 3 * F_IN:4 * F_IN])
        c = gf * c_s[...] + gi * gg
        h = go * jnp.tanh(c)
        c_s[...] = c
        h_s[...] = h

        @pl.when(d == D - 1)
        def _():
            hh = jax.nn.relu(
                jnp.dot(x_ref[...], ws_ref[...], preferred_element_type=jnp.float32)
                + jnp.dot(h, wn_ref[...], preferred_element_type=jnp.float32)
                + b1_ref[...])
            out_ref[...] = hh
            q_ref[...] = jax.nn.relu(
                jnp.dot(hh, wp_ref[...], preferred_element_type=jnp.float32)
                + bp_ref[...])

    return pl.pallas_call(
        body,
        grid=(NB, D),
        in_specs=[
            pl.BlockSpec((1, BLK, F_IN), lambda i, d: (d, i, 0)),
            pl.BlockSpec((BLK, F_IN), lambda i, d: (i, 0)),
            pl.BlockSpec((F_IN, 4 * F_IN), lambda i, d: (0, 0)),
            pl.BlockSpec((F_IN, 4 * F_IN), lambda i, d: (0, 0)),
            pl.BlockSpec((1, 4 * F_IN), lambda i, d: (0, 0)),
            pl.BlockSpec((F_IN, HID), lambda i, d: (0, 0)),
            pl.BlockSpec((F_IN, HID), lambda i, d: (0, 0)),
            pl.BlockSpec((1, HID), lambda i, d: (0, 0)),
            pl.BlockSpec((HID, HID), lambda i, d: (0, 0)),
            pl.BlockSpec((1, HID), lambda i, d: (0, 0)),
        ],
        out_specs=[
            pl.BlockSpec((BLK, HID), lambda i, d: (i, 0)),
            pl.BlockSpec((BLK, HID), lambda i, d: (i, 0)),
        ],
        out_shape=[
            jax.ShapeDtypeStruct((NP, HID), jnp.float32),
            jax.ShapeDtypeStruct((NP, HID), jnp.float32),
        ],
        scratch_shapes=[
            pltpu.VMEM((BLK, F_IN), jnp.float32),
            pltpu.VMEM((BLK, F_IN), jnp.float32),
        ],
    )(m, xp, W_ihT, W_hhT, bias, W_self1, W_neigh1, b1, W_pool, b_pool)


def _conv2(m2, h, W_self2, W_neigh2, b2):
    def body(m_ref, h_ref, ws_ref, wn_ref, b2_ref, out_ref, mx_s):
        d = pl.program_id(1)
        t = m_ref[0]
        prev = jnp.where(d == 0, jnp.zeros_like(t), mx_s[...])
        mx = jnp.maximum(t, prev)
        mx_s[...] = mx

        @pl.when(d == D - 1)
        def _():
            out_ref[...] = (
                jnp.dot(h_ref[...], ws_ref[...], preferred_element_type=jnp.float32)
                + jnp.dot(mx, wn_ref[...], preferred_element_type=jnp.float32)
                + b2_ref[...])

    return pl.pallas_call(
        body,
        grid=(NB, D),
        in_specs=[
            pl.BlockSpec((1, BLK, HID), lambda i, d: (d, i, 0)),
            pl.BlockSpec((BLK, HID), lambda i, d: (i, 0)),
            pl.BlockSpec((HID, NCLS), lambda i, d: (0, 0)),
            pl.BlockSpec((HID, NCLS), lambda i, d: (0, 0)),
            pl.BlockSpec((1, NCLS), lambda i, d: (0, 0)),
        ],
        out_specs=pl.BlockSpec((BLK, NCLS), lambda i, d: (i, 0)),
        out_shape=jax.ShapeDtypeStruct((NP, NCLS), jnp.float32),
        scratch_shapes=[pltpu.VMEM((BLK, HID), jnp.float32)],
    )(m2, h, W_self2, W_neigh2, b2)


def kernel(x, nbr, W_ih, W_hh, b_ih, b_hh, W_self1, W_neigh1, b1,
           W_pool, b_pool, W_self2, W_neigh2, b2):
    nbr = nbr.astype(jnp.int32)
    xp = jnp.pad(x, ((0, NP - N), (0, 0)))
    idx = jnp.pad(nbr.T, ((0, 0), (0, NP - N))).reshape(-1)
    per_w = (D * NP) // NW
    c1 = 16384 // F_IN
    c2 = 16384 // HID
    idx3_1 = idx.reshape(NW, per_w // c1, c1)
    idx3_2 = idx.reshape(NW, per_w // c2, c2)

    m = _sc_gather(x, idx3_1, F_IN).reshape(D, NP, F_IN)
    bias = (b_ih + b_hh).reshape(1, 4 * F_IN)
    h, q = _conv1(m, xp, W_ih.T, W_hh.T, bias, W_self1, W_neigh1,
                  b1.reshape(1, HID), W_pool, b_pool.reshape(1, HID))
    m2 = _sc_gather(q, idx3_2, HID).reshape(D, NP, HID)
    out = _conv2(m2, h, W_self2, W_neigh2, b2.reshape(1, NCLS))
    return out[:N]
